# Initial kernel scaffold; baseline (speedup 1.0000x reference)
import functools

import jax
import jax.numpy as jnp
from jax.experimental import pallas as pl
from jax.experimental.pallas import tpu as pltpu

H = 1024
I = 512
E = 8
K = 2
T = 2048
NEG = -1e30


def _silu(x):
    return x * jax.nn.sigmoid(x)


def _dense_body(x_ref, gw_ref, b_ref, w1_ref, w3_ref, w2_ref, out_ref,
                gate_scr):
    e = pl.program_id(0)

    @pl.when(e == 0)
    def _router():
        x = x_ref[...]
        logits = jax.lax.dot_general(
            x, gw_ref[...], (((1,), (1,)), ((), ())),
            preferred_element_type=jnp.float32)
        scores = jax.nn.sigmoid(logits)
        routing = scores + b_ref[...]
        iota = jax.lax.broadcasted_iota(jnp.float32, (T, E), 1)
        m1 = jnp.max(routing, axis=1, keepdims=True)
        a1 = jnp.min(jnp.where(routing == m1, iota, float(E)), axis=1,
                     keepdims=True)
        routing2 = jnp.where(iota == a1, NEG, routing)
        m2 = jnp.max(routing2, axis=1, keepdims=True)
        a2 = jnp.min(jnp.where(routing2 == m2, iota, float(E)), axis=1,
                     keepdims=True)
        s1 = jnp.sum(jnp.where(iota == a1, scores, 0.0), axis=1,
                     keepdims=True)
        s2 = jnp.sum(jnp.where(iota == a2, scores, 0.0), axis=1,
                     keepdims=True)
        denom = s1 + s2
        g = (jnp.where(iota == a1, s1, 0.0)
             + jnp.where(iota == a2, s2, 0.0)) / denom
        gate_scr[...] = g
        out_ref[...] = jnp.zeros_like(out_ref)

    x = x_ref[...]
    h1 = jax.lax.dot_general(x, w1_ref[0], (((1,), (1,)), ((), ())),
                             preferred_element_type=jnp.float32)
    h3 = jax.lax.dot_general(x, w3_ref[0], (((1,), (1,)), ((), ())),
                             preferred_element_type=jnp.float32)
    act = _silu(h1) * h3
    y = jax.lax.dot_general(act, w2_ref[0], (((1,), (1,)), ((), ())),
                            preferred_element_type=jnp.float32)
    gate_col = jnp.where(
        e < E,
        jax.lax.dynamic_slice(gate_scr[...], (0, jnp.minimum(e, E - 1)),
                              (T, 1)),
        jnp.ones((T, 1), jnp.float32))
    out_ref[...] += y * gate_col


def kernel(hidden_states, gate_w, bias, W1, W2, W3, SW1, SW2, SW3):
    orig_shape = hidden_states.shape
    x = hidden_states.reshape(T, H)
    Wa1 = jnp.concatenate([W1, SW1[None]], axis=0)
    Wa3 = jnp.concatenate([W3, SW3[None]], axis=0)
    Wa2 = jnp.concatenate([W2, SW2[None]], axis=0)
    bias2 = bias.reshape(1, E)

    out = pl.pallas_call(
        _dense_body,
        grid=(E + 1,),
        in_specs=[
            pl.BlockSpec((T, H), lambda e: (0, 0)),
            pl.BlockSpec((E, H), lambda e: (0, 0)),
            pl.BlockSpec((1, E), lambda e: (0, 0)),
            pl.BlockSpec((1, I, H), lambda e: (e, 0, 0)),
            pl.BlockSpec((1, I, H), lambda e: (e, 0, 0)),
            pl.BlockSpec((1, H, I), lambda e: (e, 0, 0)),
        ],
        out_specs=pl.BlockSpec((T, H), lambda e: (0, 0)),
        out_shape=jax.ShapeDtypeStruct((T, H), jnp.float32),
        scratch_shapes=[pltpu.VMEM((T, E), jnp.float32)],
    )(x, gate_w, bias2, Wa1, Wa3, Wa2)
    return out.reshape(orig_shape)


# speedup vs baseline: 1.7953x; 1.7953x over previous
import functools

import jax
import jax.numpy as jnp
from jax.experimental import pallas as pl
from jax.experimental.pallas import tpu as pltpu

H = 1024
I = 512
E = 8
K = 2
T = 2048
NEG = -1e30


def _silu(x):
    return x * jax.nn.sigmoid(x)


def _dense_body(x_ref, gw_ref, b_ref, w1_ref, w3_ref, w2_ref, out_ref,
                gate_scr):
    e = pl.program_id(0)

    @pl.when(e == 0)
    def _router():
        x = x_ref[...]
        logits = jax.lax.dot_general(
            x, gw_ref[...], (((1,), (1,)), ((), ())),
            preferred_element_type=jnp.float32)
        scores = jax.nn.sigmoid(logits)
        routing = scores + b_ref[...]
        iota = jax.lax.broadcasted_iota(jnp.int32, (T, E), 1)
        m1 = jnp.max(routing, axis=1, keepdims=True)
        a1 = jnp.min(jnp.where(routing == m1, iota, E), axis=1,
                     keepdims=True)
        routing2 = jnp.where(iota == a1, NEG, routing)
        m2 = jnp.max(routing2, axis=1, keepdims=True)
        a2 = jnp.min(jnp.where(routing2 == m2, iota, E), axis=1,
                     keepdims=True)
        s1 = jnp.sum(jnp.where(iota == a1, scores, 0.0), axis=1,
                     keepdims=True)
        s2 = jnp.sum(jnp.where(iota == a2, scores, 0.0), axis=1,
                     keepdims=True)
        denom = s1 + s2
        g = (jnp.where(iota == a1, s1, 0.0)
             + jnp.where(iota == a2, s2, 0.0)) / denom
        gate_scr[...] = g
        out_ref[...] = jnp.zeros_like(out_ref)

    x = x_ref[...]
    h1 = jax.lax.dot_general(x, w1_ref[0], (((1,), (1,)), ((), ())),
                             preferred_element_type=jnp.float32)
    h3 = jax.lax.dot_general(x, w3_ref[0], (((1,), (1,)), ((), ())),
                             preferred_element_type=jnp.float32)
    act = _silu(h1) * h3
    y = jax.lax.dot_general(act, w2_ref[0], (((1,), (1,)), ((), ())),
                            preferred_element_type=jnp.float32)
    lane = jax.lax.broadcasted_iota(jnp.int32, (T, E), 1)
    gate_col = jnp.where(
        e < E,
        jnp.sum(jnp.where(lane == e, gate_scr[...], 0.0), axis=1,
                keepdims=True),
        jnp.ones((T, 1), jnp.float32))
    out_ref[...] += y * gate_col


def kernel(hidden_states, gate_w, bias, W1, W2, W3, SW1, SW2, SW3):
    orig_shape = hidden_states.shape
    x = hidden_states.reshape(T, H)
    Wa1 = jnp.concatenate([W1, SW1[None]], axis=0)
    Wa3 = jnp.concatenate([W3, SW3[None]], axis=0)
    Wa2 = jnp.concatenate([W2, SW2[None]], axis=0)
    bias2 = bias.reshape(1, E)

    out = pl.pallas_call(
        _dense_body,
        grid=(E + 1,),
        in_specs=[
            pl.BlockSpec((T, H), lambda e: (0, 0)),
            pl.BlockSpec((E, H), lambda e: (0, 0)),
            pl.BlockSpec((1, E), lambda e: (0, 0)),
            pl.BlockSpec((1, I, H), lambda e: (e, 0, 0)),
            pl.BlockSpec((1, I, H), lambda e: (e, 0, 0)),
            pl.BlockSpec((1, H, I), lambda e: (e, 0, 0)),
        ],
        out_specs=pl.BlockSpec((T, H), lambda e: (0, 0)),
        out_shape=jax.ShapeDtypeStruct((T, H), jnp.float32),
        scratch_shapes=[pltpu.VMEM((T, E), jnp.float32)],
    )(x, gate_w, bias2, Wa1, Wa3, Wa2)
    return out.reshape(orig_shape)


# speedup vs baseline: 1.8020x; 1.0038x over previous
import functools

import jax
import jax.numpy as jnp
from jax.experimental import pallas as pl
from jax.experimental.pallas import tpu as pltpu

H = 1024
I = 512
E = 8
K = 2
T = 2048
NEG = -1e30


def _silu(x):
    return x * jax.nn.sigmoid(x)


def _dense_body(x_ref, gw_ref, b_ref, w1_ref, w3_ref, w2_ref, out_ref,
                gate_scr):
    e = pl.program_id(0)

    @pl.when(e == 0)
    def _router():
        x = x_ref[...]
        logits = jax.lax.dot_general(
            x, gw_ref[...], (((1,), (1,)), ((), ())),
            preferred_element_type=jnp.float32)
        scores = jax.nn.sigmoid(logits)
        routing = scores + b_ref[...]
        iota = jax.lax.broadcasted_iota(jnp.int32, (T, E), 1)
        m1 = jnp.max(routing, axis=1, keepdims=True)
        a1 = jnp.min(jnp.where(routing == m1, iota, E), axis=1,
                     keepdims=True)
        routing2 = jnp.where(iota == a1, NEG, routing)
        m2 = jnp.max(routing2, axis=1, keepdims=True)
        a2 = jnp.min(jnp.where(routing2 == m2, iota, E), axis=1,
                     keepdims=True)
        s1 = jnp.sum(jnp.where(iota == a1, scores, 0.0), axis=1,
                     keepdims=True)
        s2 = jnp.sum(jnp.where(iota == a2, scores, 0.0), axis=1,
                     keepdims=True)
        denom = s1 + s2
        g = (jnp.where(iota == a1, s1, 0.0)
             + jnp.where(iota == a2, s2, 0.0)) / denom
        gate_scr[...] = g
        out_ref[...] = jnp.zeros_like(out_ref)

    x = x_ref[...].astype(jnp.bfloat16)
    h1 = jax.lax.dot_general(x, w1_ref[0].astype(jnp.bfloat16),
                             (((1,), (1,)), ((), ())),
                             preferred_element_type=jnp.float32)
    h3 = jax.lax.dot_general(x, w3_ref[0].astype(jnp.bfloat16),
                             (((1,), (1,)), ((), ())),
                             preferred_element_type=jnp.float32)
    act = (_silu(h1) * h3).astype(jnp.bfloat16)
    y = jax.lax.dot_general(act, w2_ref[0].astype(jnp.bfloat16),
                            (((1,), (1,)), ((), ())),
                            preferred_element_type=jnp.float32)
    lane = jax.lax.broadcasted_iota(jnp.int32, (T, E), 1)
    gate_col = jnp.where(
        e < E,
        jnp.sum(jnp.where(lane == e, gate_scr[...], 0.0), axis=1,
                keepdims=True),
        jnp.ones((T, 1), jnp.float32))
    out_ref[...] += y * gate_col


def kernel(hidden_states, gate_w, bias, W1, W2, W3, SW1, SW2, SW3):
    orig_shape = hidden_states.shape
    x = hidden_states.reshape(T, H)
    Wa1 = jnp.concatenate([W1, SW1[None]], axis=0)
    Wa3 = jnp.concatenate([W3, SW3[None]], axis=0)
    Wa2 = jnp.concatenate([W2, SW2[None]], axis=0)
    bias2 = bias.reshape(1, E)

    out = pl.pallas_call(
        _dense_body,
        grid=(E + 1,),
        in_specs=[
            pl.BlockSpec((T, H), lambda e: (0, 0)),
            pl.BlockSpec((E, H), lambda e: (0, 0)),
            pl.BlockSpec((1, E), lambda e: (0, 0)),
            pl.BlockSpec((1, I, H), lambda e: (e, 0, 0)),
            pl.BlockSpec((1, I, H), lambda e: (e, 0, 0)),
            pl.BlockSpec((1, H, I), lambda e: (e, 0, 0)),
        ],
        out_specs=pl.BlockSpec((T, H), lambda e: (0, 0)),
        out_shape=jax.ShapeDtypeStruct((T, H), jnp.float32),
        scratch_shapes=[pltpu.VMEM((T, E), jnp.float32)],
    )(x, gate_w, bias2, Wa1, Wa3, Wa2)
    return out.reshape(orig_shape)


# speedup vs baseline: 1.8955x; 1.0519x over previous
import functools

import jax
import jax.numpy as jnp
from jax.experimental import pallas as pl
from jax.experimental.pallas import tpu as pltpu

H = 1024
I = 512
E = 8
K = 2
T = 2048
TB = 512
NEG = -1e30


def _silu(x):
    return x * jax.nn.sigmoid(x)


def _gate_mat(x, gw, b):
    logits = jax.lax.dot_general(x, gw, (((1,), (1,)), ((), ())),
                                 preferred_element_type=jnp.float32)
    scores = jax.nn.sigmoid(logits)
    routing = scores + b
    iota = jax.lax.broadcasted_iota(jnp.int32, (TB, E), 1)
    m1 = jnp.max(routing, axis=1, keepdims=True)
    a1 = jnp.min(jnp.where(routing == m1, iota, E), axis=1, keepdims=True)
    routing2 = jnp.where(iota == a1, NEG, routing)
    m2 = jnp.max(routing2, axis=1, keepdims=True)
    a2 = jnp.min(jnp.where(routing2 == m2, iota, E), axis=1, keepdims=True)
    s1 = jnp.sum(jnp.where(iota == a1, scores, 0.0), axis=1, keepdims=True)
    s2 = jnp.sum(jnp.where(iota == a2, scores, 0.0), axis=1, keepdims=True)
    denom = s1 + s2
    return (jnp.where(iota == a1, s1, 0.0)
            + jnp.where(iota == a2, s2, 0.0)) / denom


def _moe_body(x_ref, gw_ref, b_ref, w1_ref, w3_ref, w2_ref, out_ref):
    x = x_ref[...]
    g = _gate_mat(x, gw_ref[...], b_ref[...])
    xb = x.astype(jnp.bfloat16)
    lane = jax.lax.broadcasted_iota(jnp.int32, (TB, E), 1)
    acc = jnp.zeros((TB, H), jnp.float32)
    for e in range(E + 1):
        h1 = jax.lax.dot_general(xb, w1_ref[e], (((1,), (1,)), ((), ())),
                                 preferred_element_type=jnp.float32)
        h3 = jax.lax.dot_general(xb, w3_ref[e], (((1,), (1,)), ((), ())),
                                 preferred_element_type=jnp.float32)
        act = (_silu(h1) * h3).astype(jnp.bfloat16)
        y = jax.lax.dot_general(act, w2_ref[e], (((1,), (1,)), ((), ())),
                                preferred_element_type=jnp.float32)
        if e < E:
            gate_col = jnp.sum(jnp.where(lane == e, g, 0.0), axis=1,
                               keepdims=True)
            acc += y * gate_col
        else:
            acc += y
    out_ref[...] = acc


def kernel(hidden_states, gate_w, bias, W1, W2, W3, SW1, SW2, SW3):
    orig_shape = hidden_states.shape
    x = hidden_states.reshape(T, H)
    Wa1 = jnp.concatenate([W1, SW1[None]], axis=0).astype(jnp.bfloat16)
    Wa3 = jnp.concatenate([W3, SW3[None]], axis=0).astype(jnp.bfloat16)
    Wa2 = jnp.concatenate([W2, SW2[None]], axis=0).astype(jnp.bfloat16)
    bias2 = bias.reshape(1, E)

    out = pl.pallas_call(
        _moe_body,
        grid=(T // TB,),
        in_specs=[
            pl.BlockSpec((TB, H), lambda i: (i, 0)),
            pl.BlockSpec((E, H), lambda i: (0, 0)),
            pl.BlockSpec((1, E), lambda i: (0, 0)),
            pl.BlockSpec((E + 1, I, H), lambda i: (0, 0, 0)),
            pl.BlockSpec((E + 1, I, H), lambda i: (0, 0, 0)),
            pl.BlockSpec((E + 1, H, I), lambda i: (0, 0, 0)),
        ],
        out_specs=pl.BlockSpec((TB, H), lambda i: (i, 0)),
        out_shape=jax.ShapeDtypeStruct((T, H), jnp.float32),
    )(x, gate_w, bias2, Wa1, Wa3, Wa2)
    return out.reshape(orig_shape)


# speedup vs baseline: 2.6237x; 1.3842x over previous
import functools

import jax
import jax.numpy as jnp
from jax.experimental import pallas as pl
from jax.experimental.pallas import tpu as pltpu

H = 1024
I = 512
E = 8
K = 2
T = 2048
TB = 512
NEG = -1e30


def _silu(x):
    return x * jax.nn.sigmoid(x)


def _gate_mat(x, gw, b, rows):
    logits = jax.lax.dot_general(x, gw, (((1,), (1,)), ((), ())),
                                 preferred_element_type=jnp.float32)
    scores = jax.nn.sigmoid(logits)
    routing = scores + b
    iota = jax.lax.broadcasted_iota(jnp.int32, (rows, E), 1)
    m1 = jnp.max(routing, axis=1, keepdims=True)
    a1 = jnp.min(jnp.where(routing == m1, iota, E), axis=1, keepdims=True)
    routing2 = jnp.where(iota == a1, NEG, routing)
    m2 = jnp.max(routing2, axis=1, keepdims=True)
    a2 = jnp.min(jnp.where(routing2 == m2, iota, E), axis=1, keepdims=True)
    s1 = jnp.sum(jnp.where(iota == a1, scores, 0.0), axis=1, keepdims=True)
    s2 = jnp.sum(jnp.where(iota == a2, scores, 0.0), axis=1, keepdims=True)
    denom = s1 + s2
    return (jnp.where(iota == a1, s1, 0.0)
            + jnp.where(iota == a2, s2, 0.0)) / denom


def _moe_body(x_ref, gw_ref, b_ref, w1_ref, w3_ref, w2_ref,
              sw1_ref, sw3_ref, sw2_ref, out_ref, gate_scr):
    e = pl.program_id(0)

    @pl.when(e == 0)
    def _router():
        gate_scr[...] = _gate_mat(x_ref[...], gw_ref[...], b_ref[...], T)
        out_ref[...] = jnp.zeros_like(out_ref)

    shared = e >= E
    w1 = jnp.where(shared, sw1_ref[...], w1_ref[0])
    w3 = jnp.where(shared, sw3_ref[...], w3_ref[0])
    w2 = jnp.where(shared, sw2_ref[...], w2_ref[0])
    lane = jax.lax.broadcasted_iota(jnp.int32, (TB, E), 1)
    for j in range(T // TB):
        sl = pl.ds(j * TB, TB)
        xj = x_ref[sl, :]
        h1 = jax.lax.dot_general(xj, w1, (((1,), (1,)), ((), ())),
                                 preferred_element_type=jnp.float32)
        h3 = jax.lax.dot_general(xj, w3, (((1,), (1,)), ((), ())),
                                 preferred_element_type=jnp.float32)
        act = _silu(h1) * h3
        y = jax.lax.dot_general(act, w2, (((1,), (1,)), ((), ())),
                                preferred_element_type=jnp.float32)
        gate_col = jnp.where(
            e < E,
            jnp.sum(jnp.where(lane == jnp.minimum(e, E - 1),
                              gate_scr[sl, :], 0.0), axis=1, keepdims=True),
            jnp.ones((TB, 1), jnp.float32))
        out_ref[sl, :] += y * gate_col


def kernel(hidden_states, gate_w, bias, W1, W2, W3, SW1, SW2, SW3):
    orig_shape = hidden_states.shape
    x = hidden_states.reshape(T, H)
    bias2 = bias.reshape(1, E)

    out = pl.pallas_call(
        _moe_body,
        grid=(E + 1,),
        in_specs=[
            pl.BlockSpec((T, H), lambda e: (0, 0)),
            pl.BlockSpec((E, H), lambda e: (0, 0)),
            pl.BlockSpec((1, E), lambda e: (0, 0)),
            pl.BlockSpec((1, I, H), lambda e: (jnp.minimum(e, E - 1), 0, 0)),
            pl.BlockSpec((1, I, H), lambda e: (jnp.minimum(e, E - 1), 0, 0)),
            pl.BlockSpec((1, H, I), lambda e: (jnp.minimum(e, E - 1), 0, 0)),
            pl.BlockSpec((I, H), lambda e: (0, 0)),
            pl.BlockSpec((I, H), lambda e: (0, 0)),
            pl.BlockSpec((H, I), lambda e: (0, 0)),
        ],
        out_specs=pl.BlockSpec((T, H), lambda e: (0, 0)),
        out_shape=jax.ShapeDtypeStruct((T, H), jnp.float32),
        scratch_shapes=[pltpu.VMEM((T, E), jnp.float32)],
    )(x, gate_w, bias2, W1, W3, W2, SW1, SW3, SW2)
    return out.reshape(orig_shape)


# speedup vs baseline: 2.6333x; 1.0037x over previous
import functools

import jax
import jax.numpy as jnp
from jax.experimental import pallas as pl
from jax.experimental.pallas import tpu as pltpu

H = 1024
I = 512
E = 8
K = 2
T = 2048
TB = 512
NEG = -1e30


def _silu(x):
    return x * jax.nn.sigmoid(x)


def _gate_mat(x, gw, b, rows):
    logits = jax.lax.dot_general(x, gw, (((1,), (1,)), ((), ())),
                                 preferred_element_type=jnp.float32)
    scores = jax.nn.sigmoid(logits)
    routing = scores + b
    iota = jax.lax.broadcasted_iota(jnp.int32, (rows, E), 1)
    m1 = jnp.max(routing, axis=1, keepdims=True)
    a1 = jnp.min(jnp.where(routing == m1, iota, E), axis=1, keepdims=True)
    routing2 = jnp.where(iota == a1, NEG, routing)
    m2 = jnp.max(routing2, axis=1, keepdims=True)
    a2 = jnp.min(jnp.where(routing2 == m2, iota, E), axis=1, keepdims=True)
    s1 = jnp.sum(jnp.where(iota == a1, scores, 0.0), axis=1, keepdims=True)
    s2 = jnp.sum(jnp.where(iota == a2, scores, 0.0), axis=1, keepdims=True)
    denom = s1 + s2
    return (jnp.where(iota == a1, s1, 0.0)
            + jnp.where(iota == a2, s2, 0.0)) / denom


def _moe_body(x_ref, gw_ref, b_ref, w1_ref, w3_ref, w2_ref,
              sw1_ref, sw3_ref, sw2_ref, out_ref, gate_scr):
    e = pl.program_id(0)

    @pl.when(e == 0)
    def _router():
        gate_scr[...] = _gate_mat(x_ref[...], gw_ref[...], b_ref[...], T)
        out_ref[...] = jnp.zeros_like(out_ref)

    lane = jax.lax.broadcasted_iota(jnp.int32, (TB, E), 1)

    @pl.when(e < E)
    def _routed():
        for j in range(T // TB):
            sl = pl.ds(j * TB, TB)
            xj = x_ref[sl, :]
            h1 = jax.lax.dot_general(xj, w1_ref[0], (((1,), (1,)), ((), ())),
                                     preferred_element_type=jnp.float32)
            h3 = jax.lax.dot_general(xj, w3_ref[0], (((1,), (1,)), ((), ())),
                                     preferred_element_type=jnp.float32)
            act = _silu(h1) * h3
            y = jax.lax.dot_general(act, w2_ref[0], (((1,), (1,)), ((), ())),
                                    preferred_element_type=jnp.float32)
            gate_col = jnp.sum(
                jnp.where(lane == e, gate_scr[sl, :], 0.0), axis=1,
                keepdims=True)
            out_ref[sl, :] += y * gate_col

    @pl.when(e == E)
    def _shared():
        for j in range(T // TB):
            sl = pl.ds(j * TB, TB)
            xj = x_ref[sl, :]
            h1 = jax.lax.dot_general(xj, sw1_ref[...], (((1,), (1,)), ((), ())),
                                     preferred_element_type=jnp.float32)
            h3 = jax.lax.dot_general(xj, sw3_ref[...], (((1,), (1,)), ((), ())),
                                     preferred_element_type=jnp.float32)
            act = _silu(h1) * h3
            y = jax.lax.dot_general(act, sw2_ref[...], (((1,), (1,)), ((), ())),
                                    preferred_element_type=jnp.float32)
            out_ref[sl, :] += y


def kernel(hidden_states, gate_w, bias, W1, W2, W3, SW1, SW2, SW3):
    orig_shape = hidden_states.shape
    x = hidden_states.reshape(T, H)
    bias2 = bias.reshape(1, E)

    out = pl.pallas_call(
        _moe_body,
        grid=(E + 1,),
        in_specs=[
            pl.BlockSpec((T, H), lambda e: (0, 0)),
            pl.BlockSpec((E, H), lambda e: (0, 0)),
            pl.BlockSpec((1, E), lambda e: (0, 0)),
            pl.BlockSpec((1, I, H), lambda e: (jnp.minimum(e, E - 1), 0, 0)),
            pl.BlockSpec((1, I, H), lambda e: (jnp.minimum(e, E - 1), 0, 0)),
            pl.BlockSpec((1, H, I), lambda e: (jnp.minimum(e, E - 1), 0, 0)),
            pl.BlockSpec((I, H), lambda e: (0, 0)),
            pl.BlockSpec((I, H), lambda e: (0, 0)),
            pl.BlockSpec((H, I), lambda e: (0, 0)),
        ],
        out_specs=pl.BlockSpec((T, H), lambda e: (0, 0)),
        out_shape=jax.ShapeDtypeStruct((T, H), jnp.float32),
        scratch_shapes=[pltpu.VMEM((T, E), jnp.float32)],
    )(x, gate_w, bias2, W1, W3, W2, SW1, SW3, SW2)
    return out.reshape(orig_shape)


# speedup vs baseline: 2.7212x; 1.0334x over previous
import functools

import jax
import jax.numpy as jnp
from jax.experimental import pallas as pl
from jax.experimental.pallas import tpu as pltpu

H = 1024
I = 512
E = 8
K = 2
T = 2048
TB = 1024
NEG = -1e30


def _silu(x):
    return x * jax.nn.sigmoid(x)


def _gate_mat(x, gw, b, rows):
    logits = jax.lax.dot_general(x, gw, (((1,), (1,)), ((), ())),
                                 preferred_element_type=jnp.float32)
    scores = jax.nn.sigmoid(logits)
    routing = scores + b
    iota = jax.lax.broadcasted_iota(jnp.int32, (rows, E), 1)
    m1 = jnp.max(routing, axis=1, keepdims=True)
    a1 = jnp.min(jnp.where(routing == m1, iota, E), axis=1, keepdims=True)
    routing2 = jnp.where(iota == a1, NEG, routing)
    m2 = jnp.max(routing2, axis=1, keepdims=True)
    a2 = jnp.min(jnp.where(routing2 == m2, iota, E), axis=1, keepdims=True)
    s1 = jnp.sum(jnp.where(iota == a1, scores, 0.0), axis=1, keepdims=True)
    s2 = jnp.sum(jnp.where(iota == a2, scores, 0.0), axis=1, keepdims=True)
    denom = s1 + s2
    return (jnp.where(iota == a1, s1, 0.0)
            + jnp.where(iota == a2, s2, 0.0)) / denom


def _moe_body(x_ref, gw_ref, b_ref, w1_ref, w3_ref, w2_ref,
              sw1_ref, sw3_ref, sw2_ref, out_ref, gate_scr):
    e = pl.program_id(0)

    @pl.when(e == 0)
    def _router():
        gate_scr[...] = _gate_mat(x_ref[...], gw_ref[...], b_ref[...], T)
        out_ref[...] = jnp.zeros_like(out_ref)

    lane = jax.lax.broadcasted_iota(jnp.int32, (TB, E), 1)

    @pl.when(e < E)
    def _routed():
        for j in range(T // TB):
            sl = pl.ds(j * TB, TB)
            xj = x_ref[sl, :]
            h1 = jax.lax.dot_general(xj, w1_ref[0], (((1,), (1,)), ((), ())),
                                     preferred_element_type=jnp.float32)
            h3 = jax.lax.dot_general(xj, w3_ref[0], (((1,), (1,)), ((), ())),
                                     preferred_element_type=jnp.float32)
            act = _silu(h1) * h3
            y = jax.lax.dot_general(act, w2_ref[0], (((1,), (1,)), ((), ())),
                                    preferred_element_type=jnp.float32)
            gate_col = jnp.sum(
                jnp.where(lane == e, gate_scr[sl, :], 0.0), axis=1,
                keepdims=True)
            out_ref[sl, :] += y * gate_col

    @pl.when(e == E)
    def _shared():
        for j in range(T // TB):
            sl = pl.ds(j * TB, TB)
            xj = x_ref[sl, :]
            h1 = jax.lax.dot_general(xj, sw1_ref[...], (((1,), (1,)), ((), ())),
                                     preferred_element_type=jnp.float32)
            h3 = jax.lax.dot_general(xj, sw3_ref[...], (((1,), (1,)), ((), ())),
                                     preferred_element_type=jnp.float32)
            act = _silu(h1) * h3
            y = jax.lax.dot_general(act, sw2_ref[...], (((1,), (1,)), ((), ())),
                                    preferred_element_type=jnp.float32)
            out_ref[sl, :] += y


def kernel(hidden_states, gate_w, bias, W1, W2, W3, SW1, SW2, SW3):
    orig_shape = hidden_states.shape
    x = hidden_states.reshape(T, H)
    bias2 = bias.reshape(1, E)

    out = pl.pallas_call(
        _moe_body,
        grid=(E + 1,),
        in_specs=[
            pl.BlockSpec((T, H), lambda e: (0, 0)),
            pl.BlockSpec((E, H), lambda e: (0, 0)),
            pl.BlockSpec((1, E), lambda e: (0, 0)),
            pl.BlockSpec((1, I, H), lambda e: (jnp.minimum(e, E - 1), 0, 0)),
            pl.BlockSpec((1, I, H), lambda e: (jnp.minimum(e, E - 1), 0, 0)),
            pl.BlockSpec((1, H, I), lambda e: (jnp.minimum(e, E - 1), 0, 0)),
            pl.BlockSpec((I, H), lambda e: (0, 0)),
            pl.BlockSpec((I, H), lambda e: (0, 0)),
            pl.BlockSpec((H, I), lambda e: (0, 0)),
        ],
        out_specs=pl.BlockSpec((T, H), lambda e: (0, 0)),
        out_shape=jax.ShapeDtypeStruct((T, H), jnp.float32),
        scratch_shapes=[pltpu.VMEM((T, E), jnp.float32)],
    )(x, gate_w, bias2, W1, W3, W2, SW1, SW3, SW2)
    return out.reshape(orig_shape)
